# Initial kernel scaffold; baseline (speedup 1.0000x reference)
import jax
import jax.numpy as jnp
from jax.experimental import pallas as pl


def kernel(xyz, points, mlp, w0, b0, w1, b1, w2, b2, g0, be0, g1, be1, g2, be2, bw1, bb1, bg1, bbe1, bw2, bb2, bg2, bbe2, bw3, bb3, bg3, bbe3, bw4, bb4, bg4, bbe4):
    raise NotImplementedError("write your pallas kernel here")


# speedup vs baseline: 3.8721x; 3.8721x over previous
import functools

import jax
import jax.numpy as jnp
from jax import lax
from jax.experimental import pallas as pl
from jax.experimental.pallas import tpu as pltpu

_B = 4
_N = 2048
_S = 512
_K = 32
_R2 = 0.2 ** 2
_HW = _K * _S
_NORM = 1.0 / (_B * _HW)

_F32 = jnp.float32
_I32 = jnp.int32


def _fps_body(xyz_ref, nxyz_ref, nxyzt_ref):
    x0 = xyz_ref[:, 0, :]
    x1 = xyz_ref[:, 1, :]
    x2 = xyz_ref[:, 2, :]
    iota_n = lax.broadcasted_iota(_I32, (_B, _N), 1)
    iota_s = lax.broadcasted_iota(_I32, (_B, _S), 1)

    def body(i, carry):
        dist, far, a0, a1, a2 = carry
        oh = (iota_n == far).astype(_F32)
        c0 = jnp.sum(x0 * oh, axis=1, keepdims=True)
        c1 = jnp.sum(x1 * oh, axis=1, keepdims=True)
        c2 = jnp.sum(x2 * oh, axis=1, keepdims=True)
        mf = (iota_s == i).astype(_F32)
        a0 = a0 + c0 * mf
        a1 = a1 + c1 * mf
        a2 = a2 + c2 * mf
        d0 = x0 - c0
        d1 = x1 - c1
        d2 = x2 - c2
        d = (d0 * d0 + d1 * d1) + d2 * d2
        dist = jnp.minimum(dist, d)
        m = jnp.max(dist, axis=1, keepdims=True)
        far = jnp.min(jnp.where(dist == m, iota_n, _N), axis=1,
                      keepdims=True).astype(_I32)
        return dist, far, a0, a1, a2

    zero_s = iota_s.astype(_F32) * 0.0
    init = (x0 * 0.0 + 1e10, iota_s[:, 0:1], zero_s, zero_s * 1.0,
            zero_s * 1.0)
    _, _, a0, a1, a2 = lax.fori_loop(0, _S, body, init)

    for c, ac in enumerate((a0, a1, a2)):
        nxyz_ref[:, c, :] = ac
        nxyzt_ref[:, :, c] = ac


def _fps(xyz):
    return pl.pallas_call(
        _fps_body,
        out_shape=(jax.ShapeDtypeStruct((_B, 3, _S), _F32),
                   jax.ShapeDtypeStruct((_B, _S, 3), _F32)),
    )(xyz)


def _ball_body(p_ref, qt_ref, q_ref, f_ref, w0_ref, b0_ref, y0_ref):
    p = p_ref[0]
    qt = qt_ref[0]
    q = q_ref[0]
    f = f_ref[0]
    w0 = w0_ref[...]
    b0 = b0_ref[...]

    d = -2.0 * lax.dot(qt, p, preferred_element_type=_F32)
    qsq = jnp.sum(qt * qt, axis=1, keepdims=True)
    psq = jnp.sum(p * p, axis=0, keepdims=True)
    d = d + qsq
    d = d + psq

    maskf = (d <= _R2).astype(_F32)
    r_iota = lax.broadcasted_iota(_I32, (_N, _N), 0)
    c_iota = lax.broadcasted_iota(_I32, (_N, _N), 1)
    tri = (r_iota <= c_iota).astype(_F32)
    rank = lax.dot(maskf, tri, preferred_element_type=_F32)
    count = rank[:, _N - 1:_N]
    first = maskf * (rank == 1.0)

    fxyz = f[:3]
    fpts = f[3:]
    dims = (((1,), (1,)), ((), ()))

    def body(k, _):
        kf = k.astype(_F32)
        selk = jnp.where(count <= kf, first,
                         maskf * (rank == kf + 1.0))
        gx = lax.dot_general(fxyz, selk, dims,
                             precision=lax.Precision.HIGHEST,
                             preferred_element_type=_F32)
        gp = lax.dot_general(fpts, selk, dims,
                             preferred_element_type=_F32)
        g = jnp.concatenate([gx - q, gp], axis=0)
        y0k = lax.dot(w0, g, preferred_element_type=_F32) + b0
        y0_ref[0, :, pl.ds(k, 1), :] = y0k[:, None, :]
        return 0

    lax.fori_loop(0, _K, body, 0)


def _ball(xyz, nxyzt, nxyz, feat, w0, b0):
    return pl.pallas_call(
        _ball_body,
        grid=(_B,),
        in_specs=[
            pl.BlockSpec((1, 3, _N), lambda b: (b, 0, 0)),
            pl.BlockSpec((1, _S, 3), lambda b: (b, 0, 0)),
            pl.BlockSpec((1, 3, _S), lambda b: (b, 0, 0)),
            pl.BlockSpec((1, 67, _N), lambda b: (b, 0, 0)),
            pl.BlockSpec((128, 67), lambda b: (0, 0)),
            pl.BlockSpec((128, 1), lambda b: (0, 0)),
        ],
        out_specs=pl.BlockSpec((1, 128, _K, _S), lambda b: (b, 0, 0, 0)),
        out_shape=jax.ShapeDtypeStruct((_B, 128, _K, _S), _F32),
    )(xyz, nxyzt, nxyz, feat, w0, b0)


def _bn_in(z, st, gam, bet):
    mean = st[:, 0:1] * _NORM
    var = st[:, 1:2] * _NORM - mean * mean
    scale = gam / jnp.sqrt(var + 1e-5)
    return (z - mean[:, :, None]) * scale[:, :, None] + bet[:, :, None]


def _accum_stats(st_ref, y, is_first):
    s = jnp.sum(y, axis=1, keepdims=True)
    sq = jnp.sum(y * y, axis=1, keepdims=True)
    pair = jnp.concatenate([s, sq], axis=1)

    @pl.when(is_first)
    def _():
        st_ref[...] = pair

    @pl.when(jnp.logical_not(is_first))
    def _():
        st_ref[...] = st_ref[...] + pair


def _conv3x3_ref(hf, w, bias, out_ref):
    cout = w.shape[1]
    n_iota = lax.broadcasted_iota(_I32, (1, _HW), 1)
    w_pos = jnp.bitwise_and(n_iota, _S - 1)
    h_pos = lax.shift_right_logical(n_iota, 9)
    out_ref[0] = jnp.broadcast_to(bias[:, :, None], (cout, _K, _S))
    for di in (-1, 0, 1):
        for dj in (-1, 0, 1):
            s = (di * _S + dj) % _HW
            if s == 0:
                xs = hf
            else:
                xs = jnp.concatenate([hf[:, s:], hf[:, :s]], axis=1)
            conds = []
            if di == -1:
                conds.append(h_pos >= 1)
            if di == 1:
                conds.append(h_pos <= _K - 2)
            if dj == -1:
                conds.append(w_pos >= 1)
            if dj == 1:
                conds.append(w_pos <= _S - 2)
            if conds:
                valid = conds[0]
                for cnd in conds[1:]:
                    valid = jnp.logical_and(valid, cnd)
                xs = xs * valid.astype(_F32)
            tap = lax.dot(w[(di + 1) * 3 + dj + 1], xs,
                          preferred_element_type=_F32)
            out_ref[0] = out_ref[0] + tap.reshape(cout, _K, _S)
    return out_ref[0].reshape(cout, _HW)


def _stage_body(z_ref, st_in_ref, w_ref, bias_ref, gam_ref, bet_ref,
                out_ref, st_out_ref, *, cin, cout, conv3, has_bn):
    b = pl.program_id(0)
    z = z_ref[0]
    if has_bn:
        h = jnp.maximum(_bn_in(z, st_in_ref[...], gam_ref[...],
                               bet_ref[...]), 0.0)
    else:
        h = z
    hf = h.reshape(cin, _HW)
    bias = bias_ref[...]
    if conv3:
        y = _conv3x3_ref(hf, w_ref[...], bias, out_ref)
        _accum_stats(st_out_ref, y, b == 0)
        return
    else:
        y = lax.dot(w_ref[...], hf, preferred_element_type=_F32) + bias
    out_ref[0] = y.reshape(cout, _K, _S)
    _accum_stats(st_out_ref, y, b == 0)


def _stage(z, st_in, w, bias, gam, bet, *, cin, cout, conv3, has_bn):
    kern = functools.partial(_stage_body, cin=cin, cout=cout, conv3=conv3,
                             has_bn=has_bn)
    wspec = (pl.BlockSpec((9, cout, cin), lambda b: (0, 0, 0)) if conv3
             else pl.BlockSpec((cout, cin), lambda b: (0, 0)))
    return pl.pallas_call(
        kern,
        grid=(_B,),
        in_specs=[
            pl.BlockSpec((1, cin, _K, _S), lambda b: (b, 0, 0, 0)),
            pl.BlockSpec((cin, 2), lambda b: (0, 0)),
            wspec,
            pl.BlockSpec((cout, 1), lambda b: (0, 0)),
            pl.BlockSpec((cin, 1), lambda b: (0, 0)),
            pl.BlockSpec((cin, 1), lambda b: (0, 0)),
        ],
        out_specs=(pl.BlockSpec((1, cout, _K, _S), lambda b: (b, 0, 0, 0)),
                   pl.BlockSpec((cout, 2), lambda b: (0, 0))),
        out_shape=(jax.ShapeDtypeStruct((_B, cout, _K, _S), _F32),
                   jax.ShapeDtypeStruct((cout, 2), _F32)),
    )(z, st_in, w, bias, gam, bet)


def _resid_body(z_ref, st_ref, gam_ref, bet_ref, y0_ref, out_ref,
                st_out_ref):
    b = pl.program_id(0)
    x = _bn_in(z_ref[0], st_ref[...], gam_ref[...], bet_ref[...]) + y0_ref[0]
    out_ref[0] = x
    _accum_stats(st_out_ref, x.reshape(128, _HW), b == 0)


def _resid(z4, st4, gam, bet, y0):
    return pl.pallas_call(
        _resid_body,
        grid=(_B,),
        in_specs=[
            pl.BlockSpec((1, 128, _K, _S), lambda b: (b, 0, 0, 0)),
            pl.BlockSpec((128, 2), lambda b: (0, 0)),
            pl.BlockSpec((128, 1), lambda b: (0, 0)),
            pl.BlockSpec((128, 1), lambda b: (0, 0)),
            pl.BlockSpec((1, 128, _K, _S), lambda b: (b, 0, 0, 0)),
        ],
        out_specs=(pl.BlockSpec((1, 128, _K, _S), lambda b: (b, 0, 0, 0)),
                   pl.BlockSpec((128, 2), lambda b: (0, 0))),
        out_shape=(jax.ShapeDtypeStruct((_B, 128, _K, _S), _F32),
                   jax.ShapeDtypeStruct((128, 2), _F32)),
    )(z4, st4, gam, bet, y0)


def _final_body(z_ref, st_ref, gam_ref, bet_ref, out_ref):
    x = jnp.maximum(_bn_in(z_ref[0], st_ref[...], gam_ref[...],
                           bet_ref[...]), 0.0)
    out_ref[0] = jnp.max(x, axis=1)


def _final(z6, st6, gam, bet):
    return pl.pallas_call(
        _final_body,
        grid=(_B,),
        in_specs=[
            pl.BlockSpec((1, 256, _K, _S), lambda b: (b, 0, 0, 0)),
            pl.BlockSpec((256, 2), lambda b: (0, 0)),
            pl.BlockSpec((256, 1), lambda b: (0, 0)),
            pl.BlockSpec((256, 1), lambda b: (0, 0)),
        ],
        out_specs=pl.BlockSpec((1, 256, _S), lambda b: (b, 0, 0)),
        out_shape=jax.ShapeDtypeStruct((_B, 256, _S), _F32),
    )(z6, st6, gam, bet)


def _col(v):
    return v.reshape(-1, 1)


def _w9(w):
    return jnp.transpose(w, (2, 3, 0, 1)).reshape(9, w.shape[0], w.shape[1])


def kernel(xyz, points, mlp, w0, b0, w1, b1, w2, b2, g0, be0, g1, be1, g2,
           be2, bw1, bb1, bg1, bbe1, bw2, bb2, bg2, bbe2, bw3, bb3, bg3,
           bbe3, bw4, bb4, bg4, bbe4):
    del mlp
    nxyz, nxyzt = _fps(xyz)
    feat = jnp.concatenate([xyz, points], axis=1)
    y0 = _ball(xyz, nxyzt, nxyz, feat, w0[:, :, 0, 0], _col(b0))

    dummy = jnp.zeros((128, 2), _F32)
    zeros1 = jnp.zeros((128, 1), _F32)
    z1, s1 = _stage(y0, dummy, _w9(bw1), _col(bb1), zeros1, zeros1,
                    cin=128, cout=128, conv3=True, has_bn=False)
    z2, s2 = _stage(z1, s1, bw2[:, :, 0, 0], _col(bb2), _col(bg1), _col(bbe1),
                    cin=128, cout=64, conv3=False, has_bn=True)
    z3, s3 = _stage(z2, s2, bw3[:, :, 0, 0], _col(bb3), _col(bg2), _col(bbe2),
                    cin=64, cout=128, conv3=False, has_bn=True)
    z4, s4 = _stage(z3, s3, _w9(bw4), _col(bb4), _col(bg3), _col(bbe3),
                    cin=128, cout=128, conv3=True, has_bn=True)
    x, sx = _resid(z4, s4, _col(bg4), _col(bbe4), y0)
    z5, s5 = _stage(x, sx, w1[:, :, 0, 0], _col(b1), _col(g0), _col(be0),
                    cin=128, cout=128, conv3=False, has_bn=True)
    z6, s6 = _stage(z5, s5, w2[:, :, 0, 0], _col(b2), _col(g1), _col(be1),
                    cin=128, cout=256, conv3=False, has_bn=True)
    out = _final(z6, s6, _col(g2), _col(be2))
    return nxyz, out
